# baseline (speedup 1.0000x reference)
import functools

import jax
import jax.numpy as jnp
import numpy as np
from jax import lax
from jax.experimental import pallas as pl
from jax.experimental.pallas import tpu as pltpu
from jax.experimental.pallas import tpu_sc as plsc

_VERT_IDX = np.array([
    4646, 4779, 5143, 5109, 5112, 3207, 2391, 5398, 5786, 5925, 5831,
    5895, 2158, 6208, 6428, 6585, 6615, 6620, 2039, 2828, 6783, 7158,
    7407, 7308, 7368, 3820, 3536, 7707, 7856, 8051, 8056, 8063, 5669,
    5891, 5780, 5740, 6468, 6554, 6412, 6297, 7214, 7389, 7122, 7144,
    7975, 8059
], dtype=np.int64)

_B = 4096
_V = 8064
_C = 3
_A = 46
_L = 16
_NVT = _V // 128
_NBT = _B // 8
_NROW = _B * _V * _C // _L
_NW = 32
_B_PER_W = _B // _NW
_BT_PER_W = _B_PER_W // 8
_NCHUNK = _BT_PER_W
_NVEC = _C * 8 * _A // _L


def _build_static():
    u16 = np.unique(_VERT_IDX // _L)
    P = len(u16)
    pos_of = {int(g): p for p, g in enumerate(u16)}
    nrows = _C * 8 * P
    ndma = 8
    nd = nrows // ndma

    gidx = np.zeros((_NW, _NCHUNK, ndma, nd), dtype=np.int32)
    for w in range(_NW):
        for c in range(_NCHUNK):
            bt = w * _BT_PER_W + c
            flat = np.zeros((nrows,), dtype=np.int64)
            for s in range(nrows):
                cc, rest = divmod(s, 8 * P)
                p, bsub = divmod(rest, 8)
                vt, g = divmod(int(u16[p]), 8)
                flat[s] = (((cc * _NBT + bt) * _NVT + vt) * 8 + bsub) * 8 + g
            gidx[w, c] = flat.reshape(ndma, nd).astype(np.int32)

    srow = np.zeros((_NVEC, _L), dtype=np.int32)
    scol = np.zeros((_NVEC, _L), dtype=np.int32)
    dco = np.zeros((_NVEC, _L), dtype=np.int32)
    danc = np.zeros((_NVEC, _L), dtype=np.int32)
    dbat = np.zeros((_NVEC, _L), dtype=np.int32)
    for t in range(_NVEC * _L):
        ccbl, a = divmod(t, _A)
        cc, bl = divmod(ccbl, 8)
        v = int(_VERT_IDX[a])
        srow[t // _L, t % _L] = (cc * P + pos_of[v // _L]) * 8 + bl
        scol[t // _L, t % _L] = v % _L
        dco[t // _L, t % _L] = cc
        danc[t // _L, t % _L] = a
        dbat[t // _L, t % _L] = bl
    return gidx, srow, scol, dco, danc, dbat, nrows, ndma, nd


(_GIDX_NP, _SROW_NP, _SCOL_NP, _DCO_NP, _DANC_NP, _DBAT_NP,
 _NROWS_CHUNK, _NDMA, _ND) = _build_static()

_mesh = plsc.VectorSubcoreMesh(core_axis_name="c", subcore_axis_name="s")


@functools.partial(
    pl.kernel,
    out_type=jax.ShapeDtypeStruct((_C, _A, _B), jnp.float32),
    mesh=_mesh,
    scratch_types=[
        pltpu.VMEM((_NCHUNK, _NDMA, _ND), jnp.int32),
        pltpu.VMEM((_NVEC, _L), jnp.int32),
        pltpu.VMEM((_NVEC, _L), jnp.int32),
        pltpu.VMEM((_NVEC, _L), jnp.int32),
        pltpu.VMEM((_NVEC, _L), jnp.int32),
        pltpu.VMEM((_NVEC, _L), jnp.int32),
        pltpu.VMEM((_NROWS_CHUNK, _L), jnp.float32),
        pltpu.VMEM((_NROWS_CHUNK, _L), jnp.float32),
        pltpu.VMEM((_C, _A, 128), jnp.float32),
        pltpu.SemaphoreType.DMA,
        pltpu.SemaphoreType.DMA,
    ],
    compiler_params=pltpu.CompilerParams(use_tc_tiling_on_sc=False,
                                         needs_layout_passes=False),
)
def _gather_kernel(table, gidx, srow, scol, edco, edanc, edbat, out,
                   gidx_v, srow_v, scol_v, dco_v, danc_v, dbat_v,
                   buf0, buf1, stage, sem0, sem1):
    wid = lax.axis_index("s") * 2 + lax.axis_index("c")
    pltpu.sync_copy(gidx.at[wid], gidx_v)
    pltpu.sync_copy(srow, srow_v)
    pltpu.sync_copy(scol, scol_v)
    pltpu.sync_copy(edco, dco_v)
    pltpu.sync_copy(edanc, danc_v)
    pltpu.sync_copy(edbat, dbat_v)

    def fire(c, buf, sem):
        for d in range(_NDMA):
            pltpu.async_copy(table.at[gidx_v.at[c, d]],
                             buf.at[pl.ds(d * _ND, _ND)], sem)

    def wait(c, buf, sem):
        for d in range(_NDMA):
            pltpu.make_async_copy(table.at[gidx_v.at[c, d]],
                                  buf.at[pl.ds(d * _ND, _ND)], sem).wait()

    def extract(c, buf):
        bt8 = c * 8
        for k in range(_NVEC):
            v = plsc.load_gather(buf, [srow_v[k], scol_v[k]])
            plsc.store_scatter(
                stage, [dco_v[k], danc_v[k], dbat_v[k] + bt8], v)

    fire(0, buf0, sem0)
    fire(1, buf1, sem1)

    def body(i, carry):
        c0 = 2 * i
        c1 = c0 + 1
        wait(c0, buf0, sem0)
        extract(c0, buf0)

        @pl.when(i < _NCHUNK // 2 - 1)
        def _():
            fire(c0 + 2, buf0, sem0)

        wait(c1, buf1, sem1)
        extract(c1, buf1)

        @pl.when(i < _NCHUNK // 2 - 1)
        def _():
            fire(c1 + 2, buf1, sem1)

        return carry

    lax.fori_loop(0, _NCHUNK // 2, body, 0)

    pltpu.sync_copy(stage, out.at[:, :, pl.ds(wid * _B_PER_W, _B_PER_W)])


def kernel(vertices):
    table = (vertices.transpose(2, 0, 1)
             .reshape(_C, _NBT, 8, _NVT, 128)
             .transpose(0, 1, 3, 2, 4)
             .reshape(_NROW, _L))
    out = _gather_kernel(table, jnp.asarray(_GIDX_NP), jnp.asarray(_SROW_NP),
                         jnp.asarray(_SCOL_NP), jnp.asarray(_DCO_NP),
                         jnp.asarray(_DANC_NP), jnp.asarray(_DBAT_NP))
    return out.transpose(2, 1, 0)
